# baseline (speedup 1.0000x reference)
import functools

import jax
import jax.numpy as jnp
from jax import lax
from jax.experimental import pallas as pl
from jax.experimental.pallas import tpu as pltpu
from jax.experimental.pallas import tpu_sc as plsc

N = 10000
E = 320000
EPIECES = (160000, 160000)
D = 128
NW = 32
NCHUNK = 125
NPAD = 10240
ROWS_PER_TILE = NPAD // 16

_f32 = jnp.float32


def _make_gather(d, etot):
    ew = etot // NW
    c = ew // NCHUNK
    _MESH = plsc.VectorSubcoreMesh(core_axis_name="c", subcore_axis_name="s")

    @functools.partial(
        pl.kernel,
        mesh=_MESH,
        out_type=jax.ShapeDtypeStruct((etot, d), _f32),
        scratch_types=[
            pltpu.VMEM((NCHUNK, c), jnp.int32),
            pltpu.VMEM((NCHUNK, c), jnp.int32),
            pltpu.VMEM((c, d), _f32),
            pltpu.VMEM((c, d), _f32),
            pltpu.VMEM((c, d), _f32),
            pltpu.VMEM((c, d), _f32),
            pltpu.SemaphoreType.DMA,
            pltpu.SemaphoreType.DMA,
            pltpu.SemaphoreType.DMA,
            pltpu.SemaphoreType.DMA,
        ],
    )
    def gather(p_hbm, q_hbm, recv3d, send3d, g_hbm,
               ir_v, is_v, bpa, bqa, bpb, bqb, sem_a, sem_b, sem_wa, sem_wb):
        wid = lax.axis_index("s") * 2 + lax.axis_index("c")
        pltpu.sync_copy(recv3d.at[wid], ir_v)
        pltpu.sync_copy(send3d.at[wid], is_v)

        def fire(j, bp, bq, sem):
            cp = pltpu.async_copy(p_hbm.at[ir_v.at[j]], bp, sem)
            cq = pltpu.async_copy(q_hbm.at[is_v.at[j]], bq, sem)
            return cp, cq

        def add_rows(bp, bq):
            def row(i, carry):
                for k in range(d // 16):
                    sl = pl.ds(k * 16, 16)
                    bp[i, sl] = bp[i, sl] + bq[i, sl]
                return carry
            lax.fori_loop(0, c, row, 0)

        def fire_wb(j, bp, sem):
            base = pl.multiple_of(wid * ew + j * c, 8)
            return pltpu.async_copy(bp, g_hbm.at[pl.ds(base, c)], sem)

        def drain_wb(bp, sem):
            pltpu.make_async_copy(bp, g_hbm.at[pl.ds(0, c)], sem).wait()

        cpa, cqa = fire(0, bpa, bqa, sem_a)
        cpa.wait()
        cqa.wait()
        add_rows(bpa, bqa)
        fire_wb(0, bpa, sem_wa)

        def pair(tt, carry):
            j = 2 * tt + 1
            cpb, cqb = fire(j, bpb, bqb, sem_b)
            drain_wb(bpa, sem_wa)
            cpa2, cqa2 = fire(j + 1, bpa, bqa, sem_a)
            cpb.wait()
            cqb.wait()
            add_rows(bpb, bqb)
            fire_wb(j, bpb, sem_wb)
            cpa2.wait()
            cqa2.wait()
            add_rows(bpa, bqa)
            fire_wb(j + 1, bpa, sem_wa)
            drain_wb(bpb, sem_wb)
            return carry

        lax.fori_loop(0, (NCHUNK - 1) // 2, pair, 0)
        drain_wb(bpa, sem_wa)

    return gather


def _make_scatter(etot):
    ew = etot // NW
    c = ew // NCHUNK
    _MESH = plsc.VectorSubcoreMesh(core_axis_name="c", subcore_axis_name="s")

    @functools.partial(
        pl.kernel,
        mesh=_MESH,
        out_type=[jax.ShapeDtypeStruct((NPAD, D), _f32),
                  jax.ShapeDtypeStruct((NPAD, D), _f32)],
        scratch_types=[
            pltpu.VMEM((NCHUNK, c), jnp.int32),
            pltpu.VMEM((c, D), _f32),
            pltpu.VMEM((c, D), _f32),
            pltpu.VMEM_SHARED((NPAD, D), _f32),
            pltpu.SemaphoreType.DMA,
            pltpu.SemaphoreType.DMA,
        ],
    )
    def scatter_add(e_hbm, recv3d, zeros_hbm, out0_hbm, out1_hbm,
                    idx_v, ra, rb, acc, sem_a, sem_b):
        cc = lax.axis_index("c")
        s = lax.axis_index("s")
        wid = s * 2 + cc
        row0 = pl.multiple_of(s * ROWS_PER_TILE, 8)

        def fire_load(j, buf, sem):
            base = pl.multiple_of(wid * ew + j * c, 8)
            return pltpu.async_copy(e_hbm.at[pl.ds(base, c)], buf, sem)

        def drain_load(buf, sem):
            pltpu.make_async_copy(e_hbm.at[pl.ds(0, c)], buf, sem).wait()

        fire_load(0, ra, sem_a)
        pltpu.sync_copy(zeros_hbm.at[pl.ds(row0, ROWS_PER_TILE)],
                        acc.at[pl.ds(row0, ROWS_PER_TILE)])
        pltpu.sync_copy(recv3d.at[wid], idx_v)
        plsc.subcore_barrier()

        def pair(tt, carry):
            j = 2 * tt + 1
            fire_load(j, rb, sem_b)
            drain_load(ra, sem_a)
            pltpu.sync_copy(ra, acc.at[idx_v.at[j - 1]], add=True)
            fire_load(j + 1, ra, sem_a)
            drain_load(rb, sem_b)
            pltpu.sync_copy(rb, acc.at[idx_v.at[j]], add=True)
            return carry

        lax.fori_loop(0, (NCHUNK - 1) // 2, pair, 0)
        drain_load(ra, sem_a)
        pltpu.sync_copy(ra, acc.at[idx_v.at[NCHUNK - 1]], add=True)
        plsc.subcore_barrier()

        @pl.when(cc == 0)
        def _():
            pltpu.sync_copy(acc.at[pl.ds(row0, ROWS_PER_TILE)],
                            out0_hbm.at[pl.ds(row0, ROWS_PER_TILE)])

        @pl.when(cc == 1)
        def _():
            pltpu.sync_copy(acc.at[pl.ds(row0, ROWS_PER_TILE)],
                            out1_hbm.at[pl.ds(row0, ROWS_PER_TILE)])

    return scatter_add


_sc_cache = {}


def _sc_kernels():
    if not _sc_cache:
        for etot in set(EPIECES):
            _sc_cache['gather%d' % etot] = _make_gather(D, etot)
            _sc_cache['scatter%d' % etot] = _make_scatter(etot)
    return ([_sc_cache['gather%d' % e] for e in EPIECES],
            [_sc_cache['scatter%d' % e] for e in EPIECES])


def _ln(x, g, b):
    mu = jnp.mean(x, axis=-1, keepdims=True)
    xc = x - mu
    var = jnp.mean(xc * xc, axis=-1, keepdims=True)
    return xc * lax.rsqrt(var + 1e-5) * g + b


def _dot(a, b):
    return jnp.dot(a, b, preferred_element_type=_f32)


BE = 8000
BN = 2000


def _row_spec(rows, cols):
    return pl.BlockSpec((rows, cols), lambda i: (i, 0))


def _full_spec(rows, cols):
    return pl.BlockSpec((rows, cols), lambda i: (0, 0))


def _edge_enc_body(rel_ref, w1a, w1d, b1, w2, b2, w3, b3, g, bt, out_ref):
    rel = rel_ref[...]
    dist = jnp.sqrt(jnp.sum(rel * rel, axis=-1, keepdims=True))
    h = jnp.maximum(_dot(rel, w1a[...]) + dist * w1d[...] + b1[...], 0.0)
    h = jnp.maximum(_dot(h, w2[...]) + b2[...], 0.0)
    o = _dot(h, w3[...]) + b3[...]
    out_ref[...] = _ln(o, g[...], bt[...])


def _edge_enc(rel, w1a, w1d, b1, w2, b2, w3, b3, g, bt):
    etot = rel.shape[0]
    return pl.pallas_call(
        _edge_enc_body,
        grid=(etot // BE,),
        in_specs=[_row_spec(BE, D),
                  _full_spec(D, D), _full_spec(1, D), _full_spec(1, D),
                  _full_spec(D, D), _full_spec(1, D),
                  _full_spec(D, D), _full_spec(1, D),
                  _full_spec(1, D), _full_spec(1, D)],
        out_specs=_row_spec(BE, D),
        out_shape=jax.ShapeDtypeStruct((etot, D), _f32),
    )(rel, w1a, w1d, b1, w2, b2, w3, b3, g, bt)


def _node_enc_body(vel_ref, mat_ref, matw, matb, w1v, w1m, b1, w2, b2, w3, b3,
                   g, bt, w1r, w1s, n_out, p_out, q_out):
    oh = (mat_ref[...] == lax.broadcasted_iota(jnp.int32, (1, 8), 1)).astype(_f32)
    emb = _dot(oh, matw[...]) + matb[...]
    h = jnp.maximum(_dot(vel_ref[...], w1v[...]) + _dot(emb, w1m[...]) + b1[...], 0.0)
    h = jnp.maximum(_dot(h, w2[...]) + b2[...], 0.0)
    o = _dot(h, w3[...]) + b3[...]
    n = _ln(o, g[...], bt[...])
    n_out[...] = n
    p_out[...] = _dot(n, w1r[...])
    q_out[...] = _dot(n, w1s[...])


def _node_enc(vel16, matf, matw, matb, w1v, w1m, b1, w2, b2, w3, b3, g, bt,
              w1r, w1s):
    return pl.pallas_call(
        _node_enc_body,
        grid=(N // BN,),
        in_specs=[_row_spec(BN, 16), _row_spec(BN, 1),
                  _full_spec(8, 16), _full_spec(1, 16),
                  _full_spec(16, D), _full_spec(16, D), _full_spec(1, D),
                  _full_spec(D, D), _full_spec(1, D),
                  _full_spec(D, D), _full_spec(1, D),
                  _full_spec(1, D), _full_spec(1, D),
                  _full_spec(D, D), _full_spec(D, D)],
        out_specs=[_row_spec(BN, D)] * 3,
        out_shape=[jax.ShapeDtypeStruct((N, D), _f32)] * 3,
    )(vel16, matf, matw, matb, w1v, w1m, b1, w2, b2, w3, b3, g, bt, w1r, w1s)


def _edge_layer_body(e_ref, g_ref, w1e, b1, w2, b2, w3, b3, g, bt, out_ref):
    e = e_ref[...]
    h = jnp.maximum(_dot(e, w1e[...]) + g_ref[...] + b1[...], 0.0)
    h = jnp.maximum(_dot(h, w2[...]) + b2[...], 0.0)
    o = _dot(h, w3[...]) + b3[...]
    out_ref[...] = e + _ln(o, g[...], bt[...])


def _edge_layer(e, g_in, w1e, b1, w2, b2, w3, b3, g, bt):
    etot = e.shape[0]
    return pl.pallas_call(
        _edge_layer_body,
        grid=(etot // BE,),
        in_specs=[_row_spec(BE, D)] * 2 + [
            _full_spec(D, D), _full_spec(1, D),
            _full_spec(D, D), _full_spec(1, D),
            _full_spec(D, D), _full_spec(1, D),
            _full_spec(1, D), _full_spec(1, D)],
        out_specs=_row_spec(BE, D),
        out_shape=jax.ShapeDtypeStruct((etot, D), _f32),
    )(e, g_in, w1e, b1, w2, b2, w3, b3, g, bt)


def _node_layer(n, aggs, w1a, w1n, b1, w2, b2, w3, b3, g, bt, w1r, w1s):
    k = len(aggs)

    def body(*refs):
        n_ref = refs[0]
        a_refs = refs[1:1 + k]
        (w1a_, w1n_, b1_, w2_, b2_, w3_, b3_, g_, bt_, w1r_, w1s_,
         n_out, p_out, q_out) = refs[1 + k:]
        n_ = n_ref[...]
        agg = a_refs[0][...]
        for r in a_refs[1:]:
            agg = agg + r[...]
        h = jnp.maximum(_dot(agg, w1a_[...]) + _dot(n_, w1n_[...]) + b1_[...], 0.0)
        h = jnp.maximum(_dot(h, w2_[...]) + b2_[...], 0.0)
        o = _dot(h, w3_[...]) + b3_[...]
        nn = n_ + _ln(o, g_[...], bt_[...])
        n_out[...] = nn
        p_out[...] = _dot(nn, w1r_[...])
        q_out[...] = _dot(nn, w1s_[...])

    return pl.pallas_call(
        body,
        grid=(N // BN,),
        in_specs=[_row_spec(BN, D)] * (1 + k) + [
            _full_spec(D, D), _full_spec(D, D), _full_spec(1, D),
            _full_spec(D, D), _full_spec(1, D),
            _full_spec(D, D), _full_spec(1, D),
            _full_spec(1, D), _full_spec(1, D),
            _full_spec(D, D), _full_spec(D, D)],
        out_specs=[_row_spec(BN, D)] * 3,
        out_shape=[jax.ShapeDtypeStruct((N, D), _f32)] * 3,
    )(n, *aggs, w1a, w1n, b1, w2, b2, w3, b3, g, bt, w1r, w1s)


def _node_final(n, aggs, w1a, w1n, b1, w2, b2, w3, b3, g, bt,
                d1, d1b, d2, d2b, d3, d3b):
    k = len(aggs)

    def body(*refs):
        n_ref = refs[0]
        a_refs = refs[1:1 + k]
        (w1a_, w1n_, b1_, w2_, b2_, w3_, b3_, g_, bt_,
         d1_, d1b_, d2_, d2b_, d3_, d3b_, acc_out) = refs[1 + k:]
        n_ = n_ref[...]
        agg = a_refs[0][...]
        for r in a_refs[1:]:
            agg = agg + r[...]
        h = jnp.maximum(_dot(agg, w1a_[...]) + _dot(n_, w1n_[...]) + b1_[...], 0.0)
        h = jnp.maximum(_dot(h, w2_[...]) + b2_[...], 0.0)
        o = _dot(h, w3_[...]) + b3_[...]
        nn = n_ + _ln(o, g_[...], bt_[...])
        h = jnp.maximum(_dot(nn, d1_[...]) + d1b_[...], 0.0)
        h = jnp.maximum(_dot(h, d2_[...]) + d2b_[...], 0.0)
        acc_out[...] = _dot(h, d3_[...]) + d3b_[...]

    return pl.pallas_call(
        body,
        grid=(N // BN,),
        in_specs=[_row_spec(BN, D)] * (1 + k) + [
            _full_spec(D, D), _full_spec(D, D), _full_spec(1, D),
            _full_spec(D, D), _full_spec(1, D),
            _full_spec(D, D), _full_spec(1, D),
            _full_spec(1, D), _full_spec(1, D),
            _full_spec(D, D), _full_spec(1, D),
            _full_spec(D, D), _full_spec(1, D),
            _full_spec(D, 8), _full_spec(1, 8)],
        out_specs=_row_spec(BN, 8),
        out_shape=jax.ShapeDtypeStruct((N, 8), _f32),
    )(n, *aggs, w1a, w1n, b1, w2, b2, w3, b3, g, bt,
      d1, d1b, d2, d2b, d3, d3b)


def _row(v):
    return v.reshape(1, -1)


def kernel(materials, positions, velocities, neighbor_idxs, params):
    recv = neighbor_idxs[:, 1]
    send = neighbor_idxs[:, 2]
    recv3, send3 = [], []
    off = 0
    for etot in EPIECES:
        cw = etot // NW // NCHUNK
        recv3.append(recv[off:off + etot].reshape(NW, NCHUNK, cw))
        send3.append(send[off:off + etot].reshape(NW, NCHUNK, cw))
        off += etot

    pos128 = jnp.pad(positions[0], ((0, 0), (0, 125)))
    neg_pos128 = -pos128
    vel16 = jnp.pad(velocities.reshape(N, -1), ((0, 0), (0, 1)))
    matf = materials[0].reshape(N, 1)
    zeros_nd = jnp.zeros((NPAD, D), _f32)

    p = params
    (ne_w1, ne_b1), (ne_w2, ne_b2), (ne_w3, ne_b3) = p['node_enc']
    w1v = jnp.pad(ne_w1[:15], ((0, 1), (0, 0)))
    w1m = ne_w1[15:31]
    matw = jnp.pad(p['mat_W'], ((0, 2), (0, 0)))
    ne_g, ne_bt = p['node_enc_ln']

    (ee_w1, ee_b1), (ee_w2, ee_b2), (ee_w3, ee_b3) = p['edge_enc']
    ee_w1a = jnp.pad(ee_w1[:3], ((0, 125), (0, 0)))
    ee_w1d = _row(ee_w1[3])
    ee_g, ee_bt = p['edge_enc_ln']

    proc = p['proc']
    ew1 = [lp['edge_mlp'][0][0] for lp in proc]

    _gathers, _scatters = _sc_kernels()
    npiece = len(EPIECES)

    epieces = []
    for h in range(npiece):
        rel = _gathers[h](pos128, neg_pos128, recv3[h], send3[h])
        epieces.append(_edge_enc(rel, ee_w1a, ee_w1d, _row(ee_b1), ee_w2,
                                 _row(ee_b2), ee_w3, _row(ee_b3),
                                 _row(ee_g), _row(ee_bt)))

    nodes, pproj, qproj = _node_enc(
        vel16, matf, matw, _row(p['mat_b']), w1v, w1m, _row(ne_b1),
        ne_w2, _row(ne_b2), ne_w3, _row(ne_b3), _row(ne_g), _row(ne_bt),
        ew1[0][D:2 * D], ew1[0][2 * D:3 * D])

    (d1, d1b), (d2, d2b), (d3, d3b) = p['dec']
    d3p = jnp.pad(d3, ((0, 0), (0, 5)))
    d3bp = jnp.pad(_row(d3b), ((0, 0), (0, 5)))

    for i, lp in enumerate(proc):
        (e_w1, e_b1), (e_w2, e_b2), (e_w3, e_b3) = lp['edge_mlp']
        e_g, e_bt = lp['edge_ln']
        (n_w1, n_b1), (n_w2, n_b2), (n_w3, n_b3) = lp['node_mlp']
        n_g, n_bt = lp['node_ln']

        aggs = []
        for h in range(npiece):
            gsum = _gathers[h](pproj, qproj, recv3[h], send3[h])
            epieces[h] = _edge_layer(epieces[h], gsum, e_w1[:D], _row(e_b1),
                                     e_w2, _row(e_b2), e_w3, _row(e_b3),
                                     _row(e_g), _row(e_bt))
            agg2 = _scatters[h](epieces[h], recv3[h], zeros_nd)
            aggs += list(agg2)

        if i < len(proc) - 1:
            nodes, pproj, qproj = _node_layer(
                nodes, aggs, n_w1[:D], n_w1[D:2 * D], _row(n_b1),
                n_w2, _row(n_b2), n_w3, _row(n_b3), _row(n_g), _row(n_bt),
                ew1[i + 1][D:2 * D], ew1[i + 1][2 * D:3 * D])
        else:
            acc8 = _node_final(
                nodes, aggs, n_w1[:D], n_w1[D:2 * D], _row(n_b1),
                n_w2, _row(n_b2), n_w3, _row(n_b3), _row(n_g), _row(n_bt),
                d1, _row(d1b), d2, _row(d2b), d3p, d3bp)

    return acc8[:, :3].reshape(1, N, 3)


# speedup vs baseline: 1.0278x; 1.0278x over previous
import functools

import jax
import jax.numpy as jnp
from jax import lax
from jax.experimental import pallas as pl
from jax.experimental.pallas import tpu as pltpu
from jax.experimental.pallas import tpu_sc as plsc

N = 10000
E = 320000
EPIECES = (160000, 160000)
D = 128
NW = 32
NCHUNK = 125
NPAD = 10240
ROWS_PER_TILE = NPAD // 16

_f32 = jnp.float32


def _make_gather(d, etot):
    ew = etot // NW
    c = ew // NCHUNK
    _MESH = plsc.VectorSubcoreMesh(core_axis_name="c", subcore_axis_name="s")

    @functools.partial(
        pl.kernel,
        mesh=_MESH,
        out_type=jax.ShapeDtypeStruct((etot, d), _f32),
        scratch_types=[
            pltpu.VMEM((NCHUNK, c), jnp.int32),
            pltpu.VMEM((NCHUNK, c), jnp.int32),
            pltpu.VMEM((c, d), _f32),
            pltpu.VMEM((c, d), _f32),
            pltpu.VMEM((c, d), _f32),
            pltpu.VMEM((c, d), _f32),
            pltpu.SemaphoreType.DMA,
            pltpu.SemaphoreType.DMA,
            pltpu.SemaphoreType.DMA,
            pltpu.SemaphoreType.DMA,
        ],
    )
    def gather(p_hbm, q_hbm, recv3d, send3d, g_hbm,
               ir_v, is_v, bpa, bqa, bpb, bqb, sem_a, sem_b, sem_wa, sem_wb):
        wid = lax.axis_index("s") * 2 + lax.axis_index("c")
        pltpu.sync_copy(recv3d.at[wid], ir_v)
        pltpu.sync_copy(send3d.at[wid], is_v)

        def fire(j, bp, bq, sem):
            cp = pltpu.async_copy(p_hbm.at[ir_v.at[j]], bp, sem)
            cq = pltpu.async_copy(q_hbm.at[is_v.at[j]], bq, sem)
            return cp, cq

        def add_rows(bp, bq):
            def row(i, carry):
                for k in range(d // 16):
                    sl = pl.ds(k * 16, 16)
                    bp[i, sl] = bp[i, sl] + bq[i, sl]
                return carry
            lax.fori_loop(0, c, row, 0)

        def fire_wb(j, bp, sem):
            base = pl.multiple_of(wid * ew + j * c, 8)
            return pltpu.async_copy(bp, g_hbm.at[pl.ds(base, c)], sem)

        def drain_wb(bp, sem):
            pltpu.make_async_copy(bp, g_hbm.at[pl.ds(0, c)], sem).wait()

        cpa, cqa = fire(0, bpa, bqa, sem_a)
        cpa.wait()
        cqa.wait()
        add_rows(bpa, bqa)
        fire_wb(0, bpa, sem_wa)

        def pair(tt, carry):
            j = 2 * tt + 1
            cpb, cqb = fire(j, bpb, bqb, sem_b)
            drain_wb(bpa, sem_wa)
            cpa2, cqa2 = fire(j + 1, bpa, bqa, sem_a)
            cpb.wait()
            cqb.wait()
            add_rows(bpb, bqb)
            fire_wb(j, bpb, sem_wb)
            cpa2.wait()
            cqa2.wait()
            add_rows(bpa, bqa)
            fire_wb(j + 1, bpa, sem_wa)
            drain_wb(bpb, sem_wb)
            return carry

        lax.fori_loop(0, (NCHUNK - 1) // 2, pair, 0)
        drain_wb(bpa, sem_wa)

    return gather


def _make_scatter(etot):
    ew = etot // NW
    c = ew // NCHUNK
    _MESH = plsc.VectorSubcoreMesh(core_axis_name="c", subcore_axis_name="s")

    @functools.partial(
        pl.kernel,
        mesh=_MESH,
        out_type=[jax.ShapeDtypeStruct((NPAD, D), _f32),
                  jax.ShapeDtypeStruct((NPAD, D), _f32)],
        scratch_types=[
            pltpu.VMEM((NCHUNK, c), jnp.int32),
            pltpu.VMEM((c, D), _f32),
            pltpu.VMEM((c, D), _f32),
            pltpu.VMEM_SHARED((NPAD, D), _f32),
            pltpu.SemaphoreType.DMA,
            pltpu.SemaphoreType.DMA,
        ],
    )
    def scatter_add(e_hbm, recv3d, out0_hbm, out1_hbm,
                    idx_v, ra, rb, acc, sem_a, sem_b):
        cc = lax.axis_index("c")
        s = lax.axis_index("s")
        wid = s * 2 + cc
        row0 = pl.multiple_of(s * ROWS_PER_TILE, 8)

        def fire_load(j, buf, sem):
            base = pl.multiple_of(wid * ew + j * c, 8)
            return pltpu.async_copy(e_hbm.at[pl.ds(base, c)], buf, sem)

        def drain_load(buf, sem):
            pltpu.make_async_copy(e_hbm.at[pl.ds(0, c)], buf, sem).wait()

        fire_load(0, ra, sem_a)

        def zrow(i, carry):
            for k in range(D // 16):
                rb[i, pl.ds(k * 16, 16)] = jnp.zeros((16,), _f32)
            return carry
        lax.fori_loop(0, c, zrow, 0)
        zdescs = [pltpu.async_copy(
            rb, acc.at[pl.ds(pl.multiple_of(row0 + t * c, 8), c)], sem_b)
            for t in range(ROWS_PER_TILE // c)]
        pltpu.sync_copy(recv3d.at[wid], idx_v)
        for zd in zdescs:
            zd.wait()
        plsc.subcore_barrier()

        def pair(tt, carry):
            j = 2 * tt + 1
            fire_load(j, rb, sem_b)
            drain_load(ra, sem_a)
            pltpu.sync_copy(ra, acc.at[idx_v.at[j - 1]], add=True)
            fire_load(j + 1, ra, sem_a)
            drain_load(rb, sem_b)
            pltpu.sync_copy(rb, acc.at[idx_v.at[j]], add=True)
            return carry

        lax.fori_loop(0, (NCHUNK - 1) // 2, pair, 0)
        drain_load(ra, sem_a)
        pltpu.sync_copy(ra, acc.at[idx_v.at[NCHUNK - 1]], add=True)
        plsc.subcore_barrier()

        @pl.when(cc == 0)
        def _():
            pltpu.sync_copy(acc.at[pl.ds(row0, ROWS_PER_TILE)],
                            out0_hbm.at[pl.ds(row0, ROWS_PER_TILE)])

        @pl.when(cc == 1)
        def _():
            pltpu.sync_copy(acc.at[pl.ds(row0, ROWS_PER_TILE)],
                            out1_hbm.at[pl.ds(row0, ROWS_PER_TILE)])

    return scatter_add


_sc_cache = {}


def _sc_kernels():
    if not _sc_cache:
        for etot in set(EPIECES):
            _sc_cache['gather%d' % etot] = _make_gather(D, etot)
            _sc_cache['scatter%d' % etot] = _make_scatter(etot)
    return ([_sc_cache['gather%d' % e] for e in EPIECES],
            [_sc_cache['scatter%d' % e] for e in EPIECES])


def _ln(x, g, b):
    mu = jnp.mean(x, axis=-1, keepdims=True)
    xc = x - mu
    var = jnp.mean(xc * xc, axis=-1, keepdims=True)
    return xc * lax.rsqrt(var + 1e-5) * g + b


def _dot(a, b):
    return jnp.dot(a, b, preferred_element_type=_f32)


BE = 4000
BN = 2000


def _row_spec(rows, cols):
    return pl.BlockSpec((rows, cols), lambda i: (i, 0))


def _full_spec(rows, cols):
    return pl.BlockSpec((rows, cols), lambda i: (0, 0))


def _edge_enc_body(rel_ref, w1a, w1d, b1, w2, b2, w3, b3, g, bt, out_ref):
    rel = rel_ref[...]
    dist = jnp.sqrt(jnp.sum(rel * rel, axis=-1, keepdims=True))
    h = jnp.maximum(_dot(rel, w1a[...]) + dist * w1d[...] + b1[...], 0.0)
    h = jnp.maximum(_dot(h, w2[...]) + b2[...], 0.0)
    o = _dot(h, w3[...]) + b3[...]
    out_ref[...] = _ln(o, g[...], bt[...])


def _edge_enc(rel, w1a, w1d, b1, w2, b2, w3, b3, g, bt):
    etot = rel.shape[0]
    return pl.pallas_call(
        _edge_enc_body,
        grid=(etot // BE,),
        in_specs=[_row_spec(BE, D),
                  _full_spec(D, D), _full_spec(1, D), _full_spec(1, D),
                  _full_spec(D, D), _full_spec(1, D),
                  _full_spec(D, D), _full_spec(1, D),
                  _full_spec(1, D), _full_spec(1, D)],
        out_specs=_row_spec(BE, D),
        out_shape=jax.ShapeDtypeStruct((etot, D), _f32),
    )(rel, w1a, w1d, b1, w2, b2, w3, b3, g, bt)


def _edge_enc_l0_body(rel_ref, g0_ref, w1a, w1d, eb1, ew2, eb2, ew3, eb3,
                      eg, ebt, w1e, b1, w2, b2, w3, b3, g, bt, out_ref):
    rel = rel_ref[...]
    dist = jnp.sqrt(jnp.sum(rel * rel, axis=-1, keepdims=True))
    h = jnp.maximum(_dot(rel, w1a[...]) + dist * w1d[...] + eb1[...], 0.0)
    h = jnp.maximum(_dot(h, ew2[...]) + eb2[...], 0.0)
    e0 = _ln(_dot(h, ew3[...]) + eb3[...], eg[...], ebt[...])
    h = jnp.maximum(_dot(e0, w1e[...]) + g0_ref[...] + b1[...], 0.0)
    h = jnp.maximum(_dot(h, w2[...]) + b2[...], 0.0)
    o = _dot(h, w3[...]) + b3[...]
    out_ref[...] = e0 + _ln(o, g[...], bt[...])


def _edge_enc_l0(rel, g0, *weights):
    etot = rel.shape[0]
    return pl.pallas_call(
        _edge_enc_l0_body,
        grid=(etot // BE,),
        in_specs=[_row_spec(BE, D)] * 2 + [
            _full_spec(D, D), _full_spec(1, D), _full_spec(1, D),
            _full_spec(D, D), _full_spec(1, D),
            _full_spec(D, D), _full_spec(1, D),
            _full_spec(1, D), _full_spec(1, D),
            _full_spec(D, D), _full_spec(1, D),
            _full_spec(D, D), _full_spec(1, D),
            _full_spec(D, D), _full_spec(1, D),
            _full_spec(1, D), _full_spec(1, D)],
        out_specs=_row_spec(BE, D),
        out_shape=jax.ShapeDtypeStruct((etot, D), _f32),
    )(rel, g0, *weights)


def _node_enc_body(vel_ref, mat_ref, matw, matb, w1v, w1m, b1, w2, b2, w3, b3,
                   g, bt, w1r, w1s, n_out, p_out, q_out):
    oh = (mat_ref[...] == lax.broadcasted_iota(jnp.int32, (1, 8), 1)).astype(_f32)
    emb = _dot(oh, matw[...]) + matb[...]
    h = jnp.maximum(_dot(vel_ref[...], w1v[...]) + _dot(emb, w1m[...]) + b1[...], 0.0)
    h = jnp.maximum(_dot(h, w2[...]) + b2[...], 0.0)
    o = _dot(h, w3[...]) + b3[...]
    n = _ln(o, g[...], bt[...])
    n_out[...] = n
    p_out[...] = _dot(n, w1r[...])
    q_out[...] = _dot(n, w1s[...])


def _node_enc(vel16, matf, matw, matb, w1v, w1m, b1, w2, b2, w3, b3, g, bt,
              w1r, w1s):
    return pl.pallas_call(
        _node_enc_body,
        grid=(N // BN,),
        in_specs=[_row_spec(BN, 16), _row_spec(BN, 1),
                  _full_spec(8, 16), _full_spec(1, 16),
                  _full_spec(16, D), _full_spec(16, D), _full_spec(1, D),
                  _full_spec(D, D), _full_spec(1, D),
                  _full_spec(D, D), _full_spec(1, D),
                  _full_spec(1, D), _full_spec(1, D),
                  _full_spec(D, D), _full_spec(D, D)],
        out_specs=[_row_spec(BN, D)] * 3,
        out_shape=[jax.ShapeDtypeStruct((N, D), _f32)] * 3,
    )(vel16, matf, matw, matb, w1v, w1m, b1, w2, b2, w3, b3, g, bt, w1r, w1s)


def _edge_layer_body(e_ref, g_ref, w1e, b1, w2, b2, w3, b3, g, bt, out_ref):
    e = e_ref[...]
    h = jnp.maximum(_dot(e, w1e[...]) + g_ref[...] + b1[...], 0.0)
    h = jnp.maximum(_dot(h, w2[...]) + b2[...], 0.0)
    o = _dot(h, w3[...]) + b3[...]
    out_ref[...] = e + _ln(o, g[...], bt[...])


def _edge_layer(e, g_in, w1e, b1, w2, b2, w3, b3, g, bt):
    etot = e.shape[0]
    return pl.pallas_call(
        _edge_layer_body,
        grid=(etot // BE,),
        in_specs=[_row_spec(BE, D)] * 2 + [
            _full_spec(D, D), _full_spec(1, D),
            _full_spec(D, D), _full_spec(1, D),
            _full_spec(D, D), _full_spec(1, D),
            _full_spec(1, D), _full_spec(1, D)],
        out_specs=_row_spec(BE, D),
        out_shape=jax.ShapeDtypeStruct((etot, D), _f32),
    )(e, g_in, w1e, b1, w2, b2, w3, b3, g, bt)


def _node_layer(n, aggs, w1a, w1n, b1, w2, b2, w3, b3, g, bt, w1r, w1s):
    k = len(aggs)

    def body(*refs):
        n_ref = refs[0]
        a_refs = refs[1:1 + k]
        (w1a_, w1n_, b1_, w2_, b2_, w3_, b3_, g_, bt_, w1r_, w1s_,
         n_out, p_out, q_out) = refs[1 + k:]
        n_ = n_ref[...]
        agg = a_refs[0][...]
        for r in a_refs[1:]:
            agg = agg + r[...]
        h = jnp.maximum(_dot(agg, w1a_[...]) + _dot(n_, w1n_[...]) + b1_[...], 0.0)
        h = jnp.maximum(_dot(h, w2_[...]) + b2_[...], 0.0)
        o = _dot(h, w3_[...]) + b3_[...]
        nn = n_ + _ln(o, g_[...], bt_[...])
        n_out[...] = nn
        p_out[...] = _dot(nn, w1r_[...])
        q_out[...] = _dot(nn, w1s_[...])

    return pl.pallas_call(
        body,
        grid=(N // BN,),
        in_specs=[_row_spec(BN, D)] * (1 + k) + [
            _full_spec(D, D), _full_spec(D, D), _full_spec(1, D),
            _full_spec(D, D), _full_spec(1, D),
            _full_spec(D, D), _full_spec(1, D),
            _full_spec(1, D), _full_spec(1, D),
            _full_spec(D, D), _full_spec(D, D)],
        out_specs=[_row_spec(BN, D)] * 3,
        out_shape=[jax.ShapeDtypeStruct((N, D), _f32)] * 3,
    )(n, *aggs, w1a, w1n, b1, w2, b2, w3, b3, g, bt, w1r, w1s)


def _node_final(n, aggs, w1a, w1n, b1, w2, b2, w3, b3, g, bt,
                d1, d1b, d2, d2b, d3, d3b):
    k = len(aggs)

    def body(*refs):
        n_ref = refs[0]
        a_refs = refs[1:1 + k]
        (w1a_, w1n_, b1_, w2_, b2_, w3_, b3_, g_, bt_,
         d1_, d1b_, d2_, d2b_, d3_, d3b_, acc_out) = refs[1 + k:]
        n_ = n_ref[...]
        agg = a_refs[0][...]
        for r in a_refs[1:]:
            agg = agg + r[...]
        h = jnp.maximum(_dot(agg, w1a_[...]) + _dot(n_, w1n_[...]) + b1_[...], 0.0)
        h = jnp.maximum(_dot(h, w2_[...]) + b2_[...], 0.0)
        o = _dot(h, w3_[...]) + b3_[...]
        nn = n_ + _ln(o, g_[...], bt_[...])
        h = jnp.maximum(_dot(nn, d1_[...]) + d1b_[...], 0.0)
        h = jnp.maximum(_dot(h, d2_[...]) + d2b_[...], 0.0)
        acc_out[...] = _dot(h, d3_[...]) + d3b_[...]

    return pl.pallas_call(
        body,
        grid=(N // BN,),
        in_specs=[_row_spec(BN, D)] * (1 + k) + [
            _full_spec(D, D), _full_spec(D, D), _full_spec(1, D),
            _full_spec(D, D), _full_spec(1, D),
            _full_spec(D, D), _full_spec(1, D),
            _full_spec(1, D), _full_spec(1, D),
            _full_spec(D, D), _full_spec(1, D),
            _full_spec(D, D), _full_spec(1, D),
            _full_spec(D, 8), _full_spec(1, 8)],
        out_specs=_row_spec(BN, 8),
        out_shape=jax.ShapeDtypeStruct((N, 8), _f32),
    )(n, *aggs, w1a, w1n, b1, w2, b2, w3, b3, g, bt,
      d1, d1b, d2, d2b, d3, d3b)


def _row(v):
    return v.reshape(1, -1)


def kernel(materials, positions, velocities, neighbor_idxs, params):
    recv = neighbor_idxs[:, 1]
    send = neighbor_idxs[:, 2]
    recv3, send3 = [], []
    off = 0
    for etot in EPIECES:
        cw = etot // NW // NCHUNK
        recv3.append(recv[off:off + etot].reshape(NW, NCHUNK, cw))
        send3.append(send[off:off + etot].reshape(NW, NCHUNK, cw))
        off += etot

    pos128 = jnp.pad(positions[0], ((0, 0), (0, 125)))
    neg_pos128 = -pos128
    vel16 = jnp.pad(velocities.reshape(N, -1), ((0, 0), (0, 1)))
    matf = materials[0].reshape(N, 1)

    p = params
    (ne_w1, ne_b1), (ne_w2, ne_b2), (ne_w3, ne_b3) = p['node_enc']
    w1v = jnp.pad(ne_w1[:15], ((0, 1), (0, 0)))
    w1m = ne_w1[15:31]
    matw = jnp.pad(p['mat_W'], ((0, 2), (0, 0)))
    ne_g, ne_bt = p['node_enc_ln']

    (ee_w1, ee_b1), (ee_w2, ee_b2), (ee_w3, ee_b3) = p['edge_enc']
    ee_w1a = jnp.pad(ee_w1[:3], ((0, 125), (0, 0)))
    ee_w1d = _row(ee_w1[3])
    ee_g, ee_bt = p['edge_enc_ln']

    proc = p['proc']
    ew1 = [lp['edge_mlp'][0][0] for lp in proc]

    _gathers, _scatters = _sc_kernels()
    npiece = len(EPIECES)

    rels = [_gathers[h](pos128, neg_pos128, recv3[h], send3[h])
            for h in range(npiece)]
    epieces = [None] * npiece

    nodes, pproj, qproj = _node_enc(
        vel16, matf, matw, _row(p['mat_b']), w1v, w1m, _row(ne_b1),
        ne_w2, _row(ne_b2), ne_w3, _row(ne_b3), _row(ne_g), _row(ne_bt),
        ew1[0][D:2 * D], ew1[0][2 * D:3 * D])

    (d1, d1b), (d2, d2b), (d3, d3b) = p['dec']
    d3p = jnp.pad(d3, ((0, 0), (0, 5)))
    d3bp = jnp.pad(_row(d3b), ((0, 0), (0, 5)))

    for i, lp in enumerate(proc):
        (e_w1, e_b1), (e_w2, e_b2), (e_w3, e_b3) = lp['edge_mlp']
        e_g, e_bt = lp['edge_ln']
        (n_w1, n_b1), (n_w2, n_b2), (n_w3, n_b3) = lp['node_mlp']
        n_g, n_bt = lp['node_ln']

        aggs = []
        for h in range(npiece):
            gsum = _gathers[h](pproj, qproj, recv3[h], send3[h])
            if i == 0:
                epieces[h] = _edge_enc_l0(
                    rels[h], gsum,
                    ee_w1a, ee_w1d, _row(ee_b1), ee_w2, _row(ee_b2),
                    ee_w3, _row(ee_b3), _row(ee_g), _row(ee_bt),
                    e_w1[:D], _row(e_b1), e_w2, _row(e_b2),
                    e_w3, _row(e_b3), _row(e_g), _row(e_bt))
            else:
                epieces[h] = _edge_layer(epieces[h], gsum, e_w1[:D], _row(e_b1),
                                         e_w2, _row(e_b2), e_w3, _row(e_b3),
                                         _row(e_g), _row(e_bt))
            agg2 = _scatters[h](epieces[h], recv3[h])
            aggs += list(agg2)

        if i < len(proc) - 1:
            nodes, pproj, qproj = _node_layer(
                nodes, aggs, n_w1[:D], n_w1[D:2 * D], _row(n_b1),
                n_w2, _row(n_b2), n_w3, _row(n_b3), _row(n_g), _row(n_bt),
                ew1[i + 1][D:2 * D], ew1[i + 1][2 * D:3 * D])
        else:
            acc8 = _node_final(
                nodes, aggs, n_w1[:D], n_w1[D:2 * D], _row(n_b1),
                n_w2, _row(n_b2), n_w3, _row(n_b3), _row(n_g), _row(n_bt),
                d1, _row(d1b), d2, _row(d2b), d3p, d3bp)

    return acc8[:, :3].reshape(1, N, 3)
